# baseline (speedup 1.0000x reference)
import jax
import jax.numpy as jnp
from jax import lax
from jax.experimental import pallas as pl
from jax.experimental.pallas import tpu as pltpu
from jax.experimental.pallas import tpu_sc as plsc

N = 10000
D_IN = 128
CHUNK = 128
N_CORES = 2
N_SUBCORES = 16
N_TILES = N_CORES * N_SUBCORES
N_EDGES = 320000
CHUNKS_PER_TILE = 79
PAD_EDGES = N_TILES * CHUNKS_PER_TILE * CHUNK - N_EDGES
ACC = 16384
PAD_DST_BASE = 12288
ZCHUNK = ACC // N_SUBCORES
NOUT = 10240
ROWS_PER_TILE = 320
GROUPS = ROWS_PER_TILE // 16
WIN = 8


def _mesh():
    return plsc.VectorSubcoreMesh(
        core_axis_name="c", subcore_axis_name="s",
        num_cores=N_CORES, num_subcores=N_SUBCORES)


def _fill(ref, n, value):
    for i in range(n // 16):
        ref[pl.ds(i * 16, 16)] = jnp.full((16,), value, jnp.float32)


def _hist_body(dst_hbm, h0_hbm, h1_hbm, dst_v, ones_v, zeros_v, hist_sp,
               ssem, csem):
    c = lax.axis_index("c")
    s = lax.axis_index("s")
    wid = s * N_CORES + c
    d_idx = pltpu.async_copy(dst_hbm.at[wid], dst_v, csem)
    _fill(ones_v, CHUNK, 1.0)
    _fill(zeros_v, ZCHUNK, 0.0)
    pltpu.sync_copy(zeros_v, hist_sp.at[pl.ds(s * ZCHUNK, ZCHUNK)])
    d_idx.wait()
    plsc.subcore_barrier()
    descs = [
        pltpu.async_copy(ones_v, hist_sp.at[dst_v.at[j]], ssem, add=True)
        for j in range(CHUNKS_PER_TILE)
    ]
    for d in descs:
        d.wait()
    plsc.subcore_barrier()

    @pl.when(jnp.logical_and(s == 0, c == 0))
    def _():
        pltpu.sync_copy(hist_sp.at[pl.ds(0, NOUT)], h0_hbm)

    @pl.when(jnp.logical_and(s == 0, c == 1))
    def _():
        pltpu.sync_copy(hist_sp.at[pl.ds(0, NOUT)], h1_hbm)


def _msg_body(src_hbm, dst_hbm, y_hbm, m0_hbm, m1_hbm,
              src_v, dst_v, val_v, zeros_v, y_sp, acc_sp, gsem, ssem, csem):
    c = lax.axis_index("c")
    s = lax.axis_index("s")
    wid = s * N_CORES + c
    d_src = pltpu.async_copy(src_hbm.at[wid], src_v, csem)
    d_dst = pltpu.async_copy(dst_hbm.at[wid], dst_v, csem)
    _fill(zeros_v, ZCHUNK, 0.0)
    pltpu.sync_copy(zeros_v, acc_sp.at[pl.ds(s * ZCHUNK, ZCHUNK)])

    @pl.when(s == 0)
    def _():
        pltpu.sync_copy(y_hbm, y_sp)

    d_src.wait()
    d_dst.wait()
    plsc.subcore_barrier()

    windows = [
        list(range(w, min(w + WIN, CHUNKS_PER_TILE)))
        for w in range(0, CHUNKS_PER_TILE, WIN)
    ]

    def slot(j):
        return j % (2 * WIN)

    def fire_gathers(chunks):
        return [
            pltpu.async_copy(y_sp.at[src_v.at[j]], val_v.at[slot(j)], gsem)
            for j in chunks
        ]

    def fire_scatters(chunks):
        return [
            pltpu.async_copy(val_v.at[slot(j)], acc_sp.at[dst_v.at[j]],
                             ssem, add=True)
            for j in chunks
        ]

    gd = fire_gathers(windows[0])
    sd_prev = []
    for w, chunks in enumerate(windows):
        for d in gd:
            d.wait()
        sd = fire_scatters(chunks)
        for d in sd_prev:
            d.wait()
        sd_prev = sd
        if w + 1 < len(windows):
            gd = fire_gathers(windows[w + 1])
    for d in sd_prev:
        d.wait()
    plsc.subcore_barrier()

    @pl.when(jnp.logical_and(s == 0, c == 0))
    def _():
        pltpu.sync_copy(acc_sp.at[pl.ds(0, NOUT)], m0_hbm)

    @pl.when(jnp.logical_and(s == 0, c == 1))
    def _():
        pltpu.sync_copy(acc_sp.at[pl.ds(0, NOUT)], m1_hbm)


def _sc_hist(dstp):
    f = pl.kernel(
        _hist_body,
        out_type=(
            jax.ShapeDtypeStruct((NOUT,), jnp.float32),
            jax.ShapeDtypeStruct((NOUT,), jnp.float32),
        ),
        mesh=_mesh(),
        scratch_types=[
            pltpu.VMEM((CHUNKS_PER_TILE, CHUNK), jnp.int32),
            pltpu.VMEM((CHUNK,), jnp.float32),
            pltpu.VMEM((ZCHUNK,), jnp.float32),
            pltpu.VMEM_SHARED((ACC,), jnp.float32),
            pltpu.SemaphoreType.DMA,
            pltpu.SemaphoreType.DMA,
        ],
        compiler_params=pltpu.CompilerParams(needs_layout_passes=False),
    )
    return f(dstp)


def _sc_msg(srcp, dstp, y):
    f = pl.kernel(
        _msg_body,
        out_type=(
            jax.ShapeDtypeStruct((NOUT,), jnp.float32),
            jax.ShapeDtypeStruct((NOUT,), jnp.float32),
        ),
        mesh=_mesh(),
        scratch_types=[
            pltpu.VMEM((CHUNKS_PER_TILE, CHUNK), jnp.int32),
            pltpu.VMEM((CHUNKS_PER_TILE, CHUNK), jnp.int32),
            pltpu.VMEM((2 * WIN, CHUNK), jnp.float32),
            pltpu.VMEM((ZCHUNK,), jnp.float32),
            pltpu.VMEM_SHARED((NOUT,), jnp.float32),
            pltpu.VMEM_SHARED((ACC,), jnp.float32),
            pltpu.SemaphoreType.DMA,
            pltpu.SemaphoreType.DMA,
            pltpu.SemaphoreType.DMA,
        ],
        compiler_params=pltpu.CompilerParams(needs_layout_passes=False),
    )
    return f(srcp, dstp, y)


def _mv_body(e_ref, w_ref, x_ref):
    x_ref[...] = jnp.dot(e_ref[...], w_ref[...],
                         preferred_element_type=jnp.float32)


def _prep_body(h0_ref, h1_ref, x_ref, y_ref, dis_ref):
    deg = h0_ref[pl.ds(0, N)] + h1_ref[pl.ds(0, N)] + 1.0
    dis = lax.rsqrt(deg)
    dis_ref[...] = dis
    y_ref[pl.ds(0, N)] = x_ref[...] * dis
    y_ref[pl.ds(N, NOUT - N)] = jnp.zeros((NOUT - N,), jnp.float32)


def _out_body(m0_ref, m1_ref, y_ref, dis_ref, b_ref, o_ref):
    acc = m0_ref[pl.ds(0, N)] + m1_ref[pl.ds(0, N)] + y_ref[pl.ds(0, N)]
    o_ref[...] = dis_ref[...] * acc + b_ref[0, 0]


def kernel(edge_features, edge_index, W, b):
    src = edge_index[0].astype(jnp.int32)
    dst = edge_index[1].astype(jnp.int32)
    pad_i = jnp.arange(PAD_EDGES, dtype=jnp.int32)
    srcp = jnp.concatenate([src, pad_i % N]).reshape(
        N_TILES, CHUNKS_PER_TILE, CHUNK)
    dstp = jnp.concatenate([dst, PAD_DST_BASE + (pad_i % 4096)]).reshape(
        N_TILES, CHUNKS_PER_TILE, CHUNK)

    h0, h1 = _sc_hist(dstp)
    x2 = pl.pallas_call(
        _mv_body,
        out_shape=jax.ShapeDtypeStruct((N, 1), jnp.float32),
    )(edge_features, W)
    x = x2.reshape(N)

    y, dis = pl.pallas_call(
        _prep_body,
        out_shape=(
            jax.ShapeDtypeStruct((NOUT,), jnp.float32),
            jax.ShapeDtypeStruct((N,), jnp.float32),
        ),
    )(h0, h1, x)

    m0, m1 = _sc_msg(srcp, dstp, y)

    out = pl.pallas_call(
        _out_body,
        out_shape=jax.ShapeDtypeStruct((N,), jnp.float32),
    )(m0, m1, y, dis, b.reshape(1, 1))
    return out


# speedup vs baseline: 1.0865x; 1.0865x over previous
import jax
import jax.numpy as jnp
from jax import lax
from jax.experimental import pallas as pl
from jax.experimental.pallas import tpu as pltpu
from jax.experimental.pallas import tpu_sc as plsc

N = 10000
D_IN = 128
CHUNK = 128
N_CORES = 2
N_SUBCORES = 16
N_TILES = N_CORES * N_SUBCORES
N_EDGES = 320000
CHUNKS_PER_TILE = 79
PAD_EDGES = N_TILES * CHUNKS_PER_TILE * CHUNK - N_EDGES
ACC = 16384
PAD_DST_BASE = 12288
ZCHUNK = ACC // N_SUBCORES
NOUT = 10240
ROWS_PER_TILE = 320
GROUPS = ROWS_PER_TILE // 16
WIN = 16


def _mesh():
    return plsc.VectorSubcoreMesh(
        core_axis_name="c", subcore_axis_name="s",
        num_cores=N_CORES, num_subcores=N_SUBCORES)


def _fill(ref, n, value):
    for i in range(n // 16):
        ref[pl.ds(i * 16, 16)] = jnp.full((16,), value, jnp.float32)


def _hist_body(dst_hbm, h0_hbm, h1_hbm, dst_v, ones_v, zeros_v, hist_sp,
               ssem, csem):
    c = lax.axis_index("c")
    s = lax.axis_index("s")
    wid = s * N_CORES + c
    d_idx = pltpu.async_copy(dst_hbm.at[wid], dst_v, csem)
    _fill(ones_v, CHUNK, 1.0)
    _fill(zeros_v, ZCHUNK, 0.0)
    pltpu.sync_copy(zeros_v, hist_sp.at[pl.ds(s * ZCHUNK, ZCHUNK)])
    d_idx.wait()
    plsc.subcore_barrier()
    descs = [
        pltpu.async_copy(ones_v, hist_sp.at[dst_v.at[j]], ssem, add=True)
        for j in range(CHUNKS_PER_TILE)
    ]
    for d in descs:
        d.wait()
    plsc.subcore_barrier()

    @pl.when(jnp.logical_and(s == 0, c == 0))
    def _():
        pltpu.sync_copy(hist_sp.at[pl.ds(0, NOUT)], h0_hbm)

    @pl.when(jnp.logical_and(s == 0, c == 1))
    def _():
        pltpu.sync_copy(hist_sp.at[pl.ds(0, NOUT)], h1_hbm)


def _msg_body(src_hbm, dst_hbm, y_hbm, m0_hbm, m1_hbm,
              src_v, dst_v, val_v, zeros_v, y_sp, acc_sp, gsem, ssem, csem):
    c = lax.axis_index("c")
    s = lax.axis_index("s")
    wid = s * N_CORES + c
    d_src = pltpu.async_copy(src_hbm.at[wid], src_v, csem)
    d_dst = pltpu.async_copy(dst_hbm.at[wid], dst_v, csem)
    _fill(zeros_v, ZCHUNK, 0.0)
    pltpu.sync_copy(zeros_v, acc_sp.at[pl.ds(s * ZCHUNK, ZCHUNK)])

    @pl.when(s == 0)
    def _():
        pltpu.sync_copy(y_hbm, y_sp)

    d_src.wait()
    d_dst.wait()
    plsc.subcore_barrier()

    windows = [
        list(range(w, min(w + WIN, CHUNKS_PER_TILE)))
        for w in range(0, CHUNKS_PER_TILE, WIN)
    ]

    def slot(j):
        return j % (2 * WIN)

    def fire_gathers(chunks):
        return [
            pltpu.async_copy(y_sp.at[src_v.at[j]], val_v.at[slot(j)], gsem)
            for j in chunks
        ]

    def fire_scatters(chunks):
        return [
            pltpu.async_copy(val_v.at[slot(j)], acc_sp.at[dst_v.at[j]],
                             ssem, add=True)
            for j in chunks
        ]

    gd = fire_gathers(windows[0])
    sd_prev = []
    for w, chunks in enumerate(windows):
        for d in gd:
            d.wait()
        sd = fire_scatters(chunks)
        for d in sd_prev:
            d.wait()
        sd_prev = sd
        if w + 1 < len(windows):
            gd = fire_gathers(windows[w + 1])
    for d in sd_prev:
        d.wait()
    plsc.subcore_barrier()

    @pl.when(jnp.logical_and(s == 0, c == 0))
    def _():
        pltpu.sync_copy(acc_sp.at[pl.ds(0, NOUT)], m0_hbm)

    @pl.when(jnp.logical_and(s == 0, c == 1))
    def _():
        pltpu.sync_copy(acc_sp.at[pl.ds(0, NOUT)], m1_hbm)


def _sc_hist(dstp):
    f = pl.kernel(
        _hist_body,
        out_type=(
            jax.ShapeDtypeStruct((NOUT,), jnp.float32),
            jax.ShapeDtypeStruct((NOUT,), jnp.float32),
        ),
        mesh=_mesh(),
        scratch_types=[
            pltpu.VMEM((CHUNKS_PER_TILE, CHUNK), jnp.int32),
            pltpu.VMEM((CHUNK,), jnp.float32),
            pltpu.VMEM((ZCHUNK,), jnp.float32),
            pltpu.VMEM_SHARED((ACC,), jnp.float32),
            pltpu.SemaphoreType.DMA,
            pltpu.SemaphoreType.DMA,
        ],
        compiler_params=pltpu.CompilerParams(needs_layout_passes=False),
    )
    return f(dstp)


def _sc_msg(srcp, dstp, y):
    f = pl.kernel(
        _msg_body,
        out_type=(
            jax.ShapeDtypeStruct((NOUT,), jnp.float32),
            jax.ShapeDtypeStruct((NOUT,), jnp.float32),
        ),
        mesh=_mesh(),
        scratch_types=[
            pltpu.VMEM((CHUNKS_PER_TILE, CHUNK), jnp.int32),
            pltpu.VMEM((CHUNKS_PER_TILE, CHUNK), jnp.int32),
            pltpu.VMEM((2 * WIN, CHUNK), jnp.float32),
            pltpu.VMEM((ZCHUNK,), jnp.float32),
            pltpu.VMEM_SHARED((NOUT,), jnp.float32),
            pltpu.VMEM_SHARED((ACC,), jnp.float32),
            pltpu.SemaphoreType.DMA,
            pltpu.SemaphoreType.DMA,
            pltpu.SemaphoreType.DMA,
        ],
        compiler_params=pltpu.CompilerParams(needs_layout_passes=False),
    )
    return f(srcp, dstp, y)


def _mv_body(e_ref, w_ref, x_ref):
    x = jnp.dot(e_ref[...], w_ref[...], preferred_element_type=jnp.float32)
    x_ref[...] = x[:, 0]


def _prep_body(h0_ref, h1_ref, x_ref, y_ref, dis_ref):
    deg = h0_ref[pl.ds(0, N)] + h1_ref[pl.ds(0, N)] + 1.0
    dis = lax.rsqrt(deg)
    dis_ref[...] = dis
    y_ref[pl.ds(0, N)] = x_ref[...] * dis
    y_ref[pl.ds(N, NOUT - N)] = jnp.zeros((NOUT - N,), jnp.float32)


def _out_body(m0_ref, m1_ref, y_ref, dis_ref, b_ref, o_ref):
    acc = m0_ref[pl.ds(0, N)] + m1_ref[pl.ds(0, N)] + y_ref[pl.ds(0, N)]
    o_ref[...] = dis_ref[...] * acc + b_ref[0, 0]


def kernel(edge_features, edge_index, W, b):
    src = edge_index[0].astype(jnp.int32)
    dst = edge_index[1].astype(jnp.int32)
    pad_i = jnp.arange(PAD_EDGES, dtype=jnp.int32)
    srcp = jnp.concatenate([src, pad_i % N]).reshape(
        N_TILES, CHUNKS_PER_TILE, CHUNK)
    dstp = jnp.concatenate([dst, PAD_DST_BASE + (pad_i % 4096)]).reshape(
        N_TILES, CHUNKS_PER_TILE, CHUNK)

    h0, h1 = _sc_hist(dstp)
    x = pl.pallas_call(
        _mv_body,
        out_shape=jax.ShapeDtypeStruct((N,), jnp.float32),
    )(edge_features, W)

    y, dis = pl.pallas_call(
        _prep_body,
        out_shape=(
            jax.ShapeDtypeStruct((NOUT,), jnp.float32),
            jax.ShapeDtypeStruct((N,), jnp.float32),
        ),
    )(h0, h1, x)

    m0, m1 = _sc_msg(srcp, dstp, y)

    out = pl.pallas_call(
        _out_body,
        out_shape=jax.ShapeDtypeStruct((N,), jnp.float32),
    )(m0, m1, y, dis, b.reshape(1, 1))
    return out


# speedup vs baseline: 1.2665x; 1.1657x over previous
import jax
import jax.numpy as jnp
from jax import lax
from jax.experimental import pallas as pl
from jax.experimental.pallas import tpu as pltpu
from jax.experimental.pallas import tpu_sc as plsc

N = 10000
D_IN = 128
CHUNK = 128
N_CORES = 2
N_SUBCORES = 16
N_TILES = N_CORES * N_SUBCORES
N_EDGES = 320000
CHUNKS_PER_TILE = 79
PAD_EDGES = N_TILES * CHUNKS_PER_TILE * CHUNK - N_EDGES
EDGES_REAL = N_EDGES // N_TILES
EMAIN = 9984
ETAIL = EDGES_REAL - EMAIN
TAIL0 = N_TILES * EMAIN
PAD_PER_TILE = CHUNKS_PER_TILE * CHUNK - EDGES_REAL
ERAW = 10112
ACC = 16384
PAD_DST_BASE = 12288
ZCHUNK = ACC // N_SUBCORES
NOUT = 10240
ROWS_PER_TILE = 320
GROUPS = ROWS_PER_TILE // 16
WIN = 16


def _mesh():
    return plsc.VectorSubcoreMesh(
        core_axis_name="c", subcore_axis_name="s",
        num_cores=N_CORES, num_subcores=N_SUBCORES)


def _fill(ref, n, value):
    for i in range(n // 16):
        ref[pl.ds(i * 16, 16)] = jnp.full((16,), value, jnp.float32)


def _pad_tail(flat_ref, wid, pad_base, pad_mod):
    row16 = lax.broadcasted_iota(jnp.int32, (16,), 0)
    for i in range(PAD_PER_TILE // 16):
        pad = pad_base + (wid * PAD_PER_TILE + i * 16) % pad_mod
        flat_ref[pl.ds(EDGES_REAL + i * 16, 16)] = row16 + pad


def _stage(ei_hbm, row, wid, raw_v, csem):
    d1 = pltpu.async_copy(
        ei_hbm.at[row, pl.ds(wid * EMAIN, EMAIN)],
        raw_v.at[pl.ds(0, EMAIN)], csem)
    d2 = pltpu.async_copy(
        ei_hbm.at[row, pl.ds(TAIL0 + wid * ETAIL, ETAIL)],
        raw_v.at[pl.ds(EMAIN, ETAIL)], csem)
    return d1, d2


def _hist_body(ei_hbm, h0_hbm, h1_hbm, raw_v, dst_v, ones_v, zeros_v,
               hist_sp, ssem, csem):
    c = lax.axis_index("c")
    s = lax.axis_index("s")
    wid = s * N_CORES + c
    d1, d2 = _stage(ei_hbm, 1, wid, raw_v, csem)
    _fill(ones_v, CHUNK, 1.0)
    _fill(zeros_v, ZCHUNK, 0.0)
    pltpu.sync_copy(zeros_v, hist_sp.at[pl.ds(s * ZCHUNK, ZCHUNK)])
    d1.wait()
    d2.wait()
    _pad_tail(raw_v, wid, PAD_DST_BASE, 4096)
    for j in range(CHUNKS_PER_TILE):
        for p in range(CHUNK // 16):
            dst_v[j, pl.ds(p * 16, 16)] = raw_v[pl.ds(j * CHUNK + p * 16, 16)]
    plsc.subcore_barrier()
    descs = [
        pltpu.async_copy(ones_v, hist_sp.at[dst_v.at[j]], ssem, add=True)
        for j in range(CHUNKS_PER_TILE)
    ]
    for d in descs:
        d.wait()
    plsc.subcore_barrier()

    @pl.when(jnp.logical_and(s == 0, c == 0))
    def _():
        pltpu.sync_copy(hist_sp.at[pl.ds(0, NOUT)], h0_hbm)

    @pl.when(jnp.logical_and(s == 0, c == 1))
    def _():
        pltpu.sync_copy(hist_sp.at[pl.ds(0, NOUT)], h1_hbm)


def _msg_body(ei_hbm, y_hbm, m0_hbm, m1_hbm,
              src_v, raw_v, dst_v, val_v, zeros_v, y_sp, acc_sp,
              gsem, ssem, csem):
    c = lax.axis_index("c")
    s = lax.axis_index("s")
    wid = s * N_CORES + c
    s1, s2 = _stage(ei_hbm, 0, wid, src_v, csem)
    d1, d2 = _stage(ei_hbm, 1, wid, raw_v, csem)
    _fill(zeros_v, ZCHUNK, 0.0)
    pltpu.sync_copy(zeros_v, acc_sp.at[pl.ds(s * ZCHUNK, ZCHUNK)])

    @pl.when(s == 0)
    def _():
        pltpu.sync_copy(y_hbm, y_sp)

    s1.wait()
    s2.wait()
    _pad_tail(src_v, wid, 0, N)
    d1.wait()
    d2.wait()
    _pad_tail(raw_v, wid, PAD_DST_BASE, 4096)
    for j in range(CHUNKS_PER_TILE):
        for p in range(CHUNK // 16):
            dst_v[j, pl.ds(p * 16, 16)] = raw_v[pl.ds(j * CHUNK + p * 16, 16)]
    plsc.subcore_barrier()

    windows = [
        list(range(w, min(w + WIN, CHUNKS_PER_TILE)))
        for w in range(0, CHUNKS_PER_TILE, WIN)
    ]

    def slot(j):
        return j % (2 * WIN)

    def fire_gathers(chunks):
        return [
            pltpu.async_copy(y_sp.at[src_v.at[pl.ds(j * CHUNK, CHUNK)]],
                             val_v.at[slot(j)], gsem)
            for j in chunks
        ]

    def fire_scatters(chunks):
        return [
            pltpu.async_copy(val_v.at[slot(j)], acc_sp.at[dst_v.at[j]],
                             ssem, add=True)
            for j in chunks
        ]

    gd = fire_gathers(windows[0])
    sd_prev = []
    for w, chunks in enumerate(windows):
        for d in gd:
            d.wait()
        sd = fire_scatters(chunks)
        for d in sd_prev:
            d.wait()
        sd_prev = sd
        if w + 1 < len(windows):
            gd = fire_gathers(windows[w + 1])
    for d in sd_prev:
        d.wait()
    plsc.subcore_barrier()

    @pl.when(jnp.logical_and(s == 0, c == 0))
    def _():
        pltpu.sync_copy(acc_sp.at[pl.ds(0, NOUT)], m0_hbm)

    @pl.when(jnp.logical_and(s == 0, c == 1))
    def _():
        pltpu.sync_copy(acc_sp.at[pl.ds(0, NOUT)], m1_hbm)


def _sc_hist(ei):
    f = pl.kernel(
        _hist_body,
        out_type=(
            jax.ShapeDtypeStruct((NOUT,), jnp.float32),
            jax.ShapeDtypeStruct((NOUT,), jnp.float32),
        ),
        mesh=_mesh(),
        scratch_types=[
            pltpu.VMEM((ERAW,), jnp.int32),
            pltpu.VMEM((CHUNKS_PER_TILE, CHUNK), jnp.int32),
            pltpu.VMEM((CHUNK,), jnp.float32),
            pltpu.VMEM((ZCHUNK,), jnp.float32),
            pltpu.VMEM_SHARED((ACC,), jnp.float32),
            pltpu.SemaphoreType.DMA,
            pltpu.SemaphoreType.DMA,
        ],
        compiler_params=pltpu.CompilerParams(needs_layout_passes=False),
    )
    return f(ei)


def _sc_msg(ei, y):
    f = pl.kernel(
        _msg_body,
        out_type=(
            jax.ShapeDtypeStruct((NOUT,), jnp.float32),
            jax.ShapeDtypeStruct((NOUT,), jnp.float32),
        ),
        mesh=_mesh(),
        scratch_types=[
            pltpu.VMEM((ERAW,), jnp.int32),
            pltpu.VMEM((ERAW,), jnp.int32),
            pltpu.VMEM((CHUNKS_PER_TILE, CHUNK), jnp.int32),
            pltpu.VMEM((2 * WIN, CHUNK), jnp.float32),
            pltpu.VMEM((ZCHUNK,), jnp.float32),
            pltpu.VMEM_SHARED((NOUT,), jnp.float32),
            pltpu.VMEM_SHARED((ACC,), jnp.float32),
            pltpu.SemaphoreType.DMA,
            pltpu.SemaphoreType.DMA,
            pltpu.SemaphoreType.DMA,
        ],
        compiler_params=pltpu.CompilerParams(needs_layout_passes=False),
    )
    return f(ei, y)


def _mv_body(e_ref, w_ref, x_ref):
    x = jnp.dot(e_ref[...], w_ref[...], preferred_element_type=jnp.float32)
    x_ref[...] = x[:, 0]


def _prep_body(h0_ref, h1_ref, x_ref, y_ref, dis_ref):
    deg = h0_ref[pl.ds(0, N)] + h1_ref[pl.ds(0, N)] + 1.0
    dis = lax.rsqrt(deg)
    dis_ref[...] = dis
    y_ref[pl.ds(0, N)] = x_ref[...] * dis
    y_ref[pl.ds(N, NOUT - N)] = jnp.zeros((NOUT - N,), jnp.float32)


def _out_body(m0_ref, m1_ref, y_ref, dis_ref, b_ref, o_ref):
    acc = m0_ref[pl.ds(0, N)] + m1_ref[pl.ds(0, N)] + y_ref[pl.ds(0, N)]
    o_ref[...] = dis_ref[...] * acc + b_ref[0, 0]


def kernel(edge_features, edge_index, W, b):
    ei = edge_index.astype(jnp.int32)

    h0, h1 = _sc_hist(ei)
    x = pl.pallas_call(
        _mv_body,
        out_shape=jax.ShapeDtypeStruct((N,), jnp.float32),
    )(edge_features, W)

    y, dis = pl.pallas_call(
        _prep_body,
        out_shape=(
            jax.ShapeDtypeStruct((NOUT,), jnp.float32),
            jax.ShapeDtypeStruct((N,), jnp.float32),
        ),
    )(h0, h1, x)

    m0, m1 = _sc_msg(ei, y)

    out = pl.pallas_call(
        _out_body,
        out_shape=jax.ShapeDtypeStruct((N,), jnp.float32),
    )(m0, m1, y, dis, b.reshape(1, 1))
    return out
